# baseline (speedup 1.0000x reference)
import functools

import jax
import jax.numpy as jnp
from jax import lax
from jax.experimental import pallas as pl
from jax.experimental.pallas import tpu as pltpu
from jax.experimental.pallas import tpu_sc as plsc

NW = 32
DW = 128
ROWS = 128
NBUF = 5


def _scatter_transpose(buf, trans, width):
    iot = lax.iota(jnp.int32, 16)

    def row_body(d, carry):
        for k in range(width // 16):
            v = buf[d, pl.ds(k * 16, 16)]
            plsc.store_scatter(trans, [iot + k * 16, jnp.full((16,), d, jnp.int32)], v)
        return carry

    lax.fori_loop(0, 64, row_body, 0)


def kernel(x, table):
    B, H = x.shape
    V, D = table.shape
    total = B * H
    per_w = total // NW
    n_ch = per_w // ROWS

    t_T = table.T
    n_full = V // DW
    n_even = (n_full // NW) * NW
    tail = V - n_full * DW
    tail64 = jnp.pad(lax.slice(table, (n_full * DW, 0), (V, D)),
                     ((0, 0), (0, DW - D)))
    x3 = x.reshape(NW, per_w // DW, DW).astype(jnp.int32)
    mesh = plsc.VectorSubcoreMesh(core_axis_name="c", subcore_axis_name="s")

    @functools.partial(
        pl.kernel,
        out_type=jax.ShapeDtypeStruct((V, DW), jnp.float32),
        mesh=mesh,
        scratch_types=[
            pltpu.VMEM((64, DW), jnp.float32),
            pltpu.VMEM((tail, DW), jnp.float32),
            pltpu.VMEM((DW, DW), jnp.float32),
        ],
        compiler_params=pltpu.CompilerParams(use_tc_tiling_on_sc=True, needs_layout_passes=False),
    )
    def transpose_k(t_hbm, tail_hbm, wide_hbm, buf, tbuf, trans):
        wid = lax.axis_index("s") * 2 + lax.axis_index("c")

        def panel(c):
            pltpu.sync_copy(t_hbm.at[:, pl.ds(c, DW)], buf)
            _scatter_transpose(buf, trans, DW)
            pltpu.sync_copy(trans, wide_hbm.at[pl.ds(c, DW)])

        def group(g, carry):
            panel((g * NW + wid) * DW)
            return carry

        lax.fori_loop(0, n_even // NW, group, 0)
        for p in range(n_even, n_full):
            @pl.when(wid == (p - n_even))
            def _():
                panel(p * DW)

        @pl.when(wid == (n_full - n_even))
        def _():
            pltpu.sync_copy(tail_hbm, tbuf)
            pltpu.sync_copy(tbuf, wide_hbm.at[pl.ds(n_full * DW, tail)])

    @functools.partial(
        pl.kernel,
        out_type=jax.ShapeDtypeStruct((total, DW), jnp.float32),
        mesh=mesh,
        scratch_types=[
            pltpu.VMEM((per_w // DW, DW), jnp.int32),
            pltpu.VMEM((NBUF, ROWS, DW), jnp.float32),
            pltpu.SemaphoreType.DMA,
            pltpu.SemaphoreType.DMA,
        ],
        compiler_params=pltpu.CompilerParams(use_tc_tiling_on_sc=True, needs_layout_passes=False),
    )
    def gather_k(x_hbm, table_hbm, out_hbm, idx_v, buf, sem_g, sem_w):
        wid = lax.axis_index("s") * 2 + lax.axis_index("c")
        base = wid * per_w
        pltpu.sync_copy(x_hbm.at[wid], idx_v)

        def group(g, carry):
            gathers = []
            for b in range(NBUF):
                j = g * NBUF + b
                gathers.append(
                    pltpu.async_copy(table_hbm.at[idx_v.at[j]], buf.at[b], sem_g)
                )
            writes = []
            for b in range(NBUF):
                j = g * NBUF + b
                gathers[b].wait()
                writes.append(
                    pltpu.async_copy(
                        buf.at[b], out_hbm.at[pl.ds(base + j * ROWS, ROWS)], sem_w
                    )
                )
            for b in range(NBUF):
                writes[b].wait()
            return carry

        lax.fori_loop(0, n_ch // NBUF, group, 0)

    wide = transpose_k(t_T, tail64)
    out = gather_k(x3, wide)
    return out[:, :D].reshape(B, H, D)


# speedup vs baseline: 1.0014x; 1.0014x over previous
import functools

import jax
import jax.numpy as jnp
from jax import lax
from jax.experimental import pallas as pl
from jax.experimental.pallas import tpu as pltpu
from jax.experimental.pallas import tpu_sc as plsc

NW = 32
DW = 128
ROWS = 128
NBUF = 5


def _scatter_transpose(buf, trans, width):
    rows = [lax.iota(jnp.int32, 16) + k * 16 for k in range(width // 16)]

    def row_body(g, carry):
        for dd in range(8):
            d = g * 8 + dd
            dvec = jnp.full((16,), d, jnp.int32)
            for k in range(width // 16):
                v = buf[d, pl.ds(k * 16, 16)]
                plsc.store_scatter(trans, [rows[k], dvec], v)
        return carry

    lax.fori_loop(0, 8, row_body, 0)


def kernel(x, table):
    B, H = x.shape
    V, D = table.shape
    total = B * H
    per_w = total // NW
    n_ch = per_w // ROWS

    t_T = table.T
    n_full = V // DW
    n_even = (n_full // NW) * NW
    tail = V - n_full * DW
    tail64 = jnp.pad(lax.slice(table, (n_full * DW, 0), (V, D)),
                     ((0, 0), (0, DW - D)))
    x3 = x.reshape(NW, per_w // DW, DW).astype(jnp.int32)
    mesh = plsc.VectorSubcoreMesh(core_axis_name="c", subcore_axis_name="s")

    @functools.partial(
        pl.kernel,
        out_type=jax.ShapeDtypeStruct((V, DW), jnp.float32),
        mesh=mesh,
        scratch_types=[
            pltpu.VMEM((64, DW), jnp.float32),
            pltpu.VMEM((tail, DW), jnp.float32),
            pltpu.VMEM((DW, DW), jnp.float32),
        ],
        compiler_params=pltpu.CompilerParams(use_tc_tiling_on_sc=True, needs_layout_passes=False),
    )
    def transpose_k(t_hbm, tail_hbm, wide_hbm, buf, tbuf, trans):
        wid = lax.axis_index("s") * 2 + lax.axis_index("c")

        def panel(c):
            pltpu.sync_copy(t_hbm.at[:, pl.ds(c, DW)], buf)
            _scatter_transpose(buf, trans, DW)
            pltpu.sync_copy(trans, wide_hbm.at[pl.ds(c, DW)])

        def group(g, carry):
            panel((g * NW + wid) * DW)
            return carry

        lax.fori_loop(0, n_even // NW, group, 0)
        for p in range(n_even, n_full):
            @pl.when(wid == (p - n_even))
            def _():
                panel(p * DW)

        @pl.when(wid == (n_full - n_even))
        def _():
            pltpu.sync_copy(tail_hbm, tbuf)
            pltpu.sync_copy(tbuf, wide_hbm.at[pl.ds(n_full * DW, tail)])

    @functools.partial(
        pl.kernel,
        out_type=jax.ShapeDtypeStruct((total, DW), jnp.float32),
        mesh=mesh,
        scratch_types=[
            pltpu.VMEM((per_w // DW, DW), jnp.int32),
            pltpu.VMEM((NBUF, ROWS, DW), jnp.float32),
            pltpu.SemaphoreType.DMA,
            pltpu.SemaphoreType.DMA,
        ],
        compiler_params=pltpu.CompilerParams(use_tc_tiling_on_sc=True, needs_layout_passes=False),
    )
    def gather_k(x_hbm, table_hbm, out_hbm, idx_v, buf, sem_g, sem_w):
        wid = lax.axis_index("s") * 2 + lax.axis_index("c")
        base = wid * per_w
        pltpu.sync_copy(x_hbm.at[wid], idx_v)

        def group(g, carry):
            gathers = []
            for b in range(NBUF):
                j = g * NBUF + b
                gathers.append(
                    pltpu.async_copy(table_hbm.at[idx_v.at[j]], buf.at[b], sem_g)
                )
            writes = []
            for b in range(NBUF):
                j = g * NBUF + b
                gathers[b].wait()
                writes.append(
                    pltpu.async_copy(
                        buf.at[b], out_hbm.at[pl.ds(base + j * ROWS, ROWS)], sem_w
                    )
                )
            for b in range(NBUF):
                writes[b].wait()
            return carry

        lax.fori_loop(0, n_ch // NBUF, group, 0)

    wide = transpose_k(t_T, tail64)
    out = gather_k(x3, wide)
    return out[:, :D].reshape(B, H, D)


# speedup vs baseline: 2.2823x; 2.2791x over previous
import functools

import jax
import jax.numpy as jnp
from jax import lax
from jax.experimental import pallas as pl
from jax.experimental.pallas import tpu as pltpu
from jax.experimental.pallas import tpu_sc as plsc

NW = 32
DW = 128
ROWS = 256
NBUF = 5


def kernel(x, table):
    B, H = x.shape
    V, D = table.shape
    total = B * H
    per_w = total // NW
    n_ch = per_w // ROWS

    x_resh = x.reshape(NW, per_w).astype(jnp.int32)
    mesh = plsc.VectorSubcoreMesh(core_axis_name="c", subcore_axis_name="s")

    @functools.partial(
        pl.kernel,
        out_type=jax.ShapeDtypeStruct((total, DW), jnp.float32),
        mesh=mesh,
        scratch_types=[
            pltpu.VMEM((per_w,), jnp.int32),
            pltpu.VMEM((NBUF, ROWS, D), jnp.float32),
            pltpu.SemaphoreType.DMA,
            pltpu.SemaphoreType.DMA,
        ],
        compiler_params=pltpu.CompilerParams(use_tc_tiling_on_sc=False),
    )
    def emb(x_hbm, table_hbm, out_hbm, idx_v, buf, sem_g, sem_w):
        wid = lax.axis_index("s") * 2 + lax.axis_index("c")
        base = wid * per_w
        pltpu.sync_copy(x_hbm.at[wid], idx_v)

        def group(g, carry):
            gathers = []
            for b in range(NBUF):
                j = (g * NBUF + b) * ROWS
                gathers.append(
                    pltpu.async_copy(
                        table_hbm.at[idx_v.at[pl.ds(j, ROWS)]], buf.at[b], sem_g
                    )
                )
            writes = []
            for b in range(NBUF):
                j = (g * NBUF + b) * ROWS
                gathers[b].wait()
                writes.append(
                    pltpu.async_copy(
                        buf.at[b],
                        out_hbm.at[pl.ds(base + j, ROWS), pl.ds(0, D)],
                        sem_w,
                    )
                )
            for b in range(NBUF):
                writes[b].wait()
            return carry

        lax.fori_loop(0, n_ch // NBUF, group, 0)

    out = emb(x_resh, table)
    return out[:, :D].reshape(B, H, D)
